# baseline (speedup 1.0000x reference)
import jax
import jax.numpy as jnp
import numpy as np
from jax import lax
from jax.experimental import pallas as pl

_N = 6400000
_LANES = 1280
_ROWS = _N // _LANES
_BLOCK_ROWS = 1000
_GRID = _ROWS // _BLOCK_ROWS

_KEEP_PROB = 0.9
_INV_KEEP = np.float32(1.0 / _KEEP_PROB)

_KD0 = 2247515013
_KD1 = 2545468385
_K0 = np.int32(np.uint32(_KD0))
_K1 = np.int32(np.uint32(_KD1))
_K2 = np.int32(np.uint32((_KD0 ^ _KD1 ^ 0x1BD11BDA) & 0xFFFFFFFF))
_KS = (_K0, _K1, _K2)
_ROTS = ((13, 15, 26, 6), (17, 29, 16, 24))
_THRESH = np.int32(838861 << 9)


def _rotl(x, r):
    return lax.shift_left(x, np.int32(r)) | lax.shift_right_logical(
        x, np.int32(32 - r))


_CHUNK_ROWS = 8


def _cipher_bits(idx):
    x0 = _K0
    x1 = idx + _K1
    for i in range(5):
        for r in _ROTS[i % 2]:
            x0 = x0 + x1
            x1 = _rotl(x1, r) ^ x0
        x0 = x0 + _KS[(i + 1) % 3]
        x1 = x1 + _KS[(i + 2) % 3] + np.int32(i + 1)
    return x0 ^ x1


def _dropout_block(v_ref, o_ref):
    pid = pl.program_id(0)
    base = pid * np.int32(_BLOCK_ROWS * _LANES)
    idx0 = (base
            + lax.broadcasted_iota(jnp.int32, (_CHUNK_ROWS, _LANES), 0)
            * np.int32(_LANES)
            + lax.broadcasted_iota(jnp.int32, (_CHUNK_ROWS, _LANES), 1))

    def body(j, idx):
        r0 = pl.multiple_of(j * _CHUNK_ROWS, _CHUNK_ROWS)
        bits = _cipher_bits(idx)
        keep = (bits < 0) | (bits >= _THRESH)
        o_ref[pl.ds(r0, _CHUNK_ROWS), :] = jnp.where(
            keep, v_ref[pl.ds(r0, _CHUNK_ROWS), :] * _INV_KEEP,
            np.float32(0.0))
        return idx + np.int32(_CHUNK_ROWS * _LANES)

    lax.fori_loop(0, _BLOCK_ROWS // _CHUNK_ROWS, body, idx0)


def kernel(indices, values):
    v2d = values.reshape(_ROWS, _LANES)
    out = pl.pallas_call(
        _dropout_block,
        grid=(_GRID,),
        in_specs=[pl.BlockSpec((_BLOCK_ROWS, _LANES), lambda i: (i, 0))],
        out_specs=pl.BlockSpec((_BLOCK_ROWS, _LANES), lambda i: (i, 0)),
        out_shape=jax.ShapeDtypeStruct((_ROWS, _LANES), jnp.float32),
    )(v2d)
    return indices, out.reshape(_N)


# speedup vs baseline: 1.5162x; 1.5162x over previous
import jax
import jax.numpy as jnp
import numpy as np
from jax import lax
from jax.experimental import pallas as pl

_N = 6400000
_LANES = 128
_ROWS = _N // _LANES
_BLOCK_ROWS = 2000
_GRID = _ROWS // _BLOCK_ROWS

_KEEP_PROB = 0.9
_INV_KEEP = np.float32(1.0 / _KEEP_PROB)

_KD0 = 2247515013
_KD1 = 2545468385
_K0 = np.int32(np.uint32(_KD0))
_K1 = np.int32(np.uint32(_KD1))
_K2 = np.int32(np.uint32((_KD0 ^ _KD1 ^ 0x1BD11BDA) & 0xFFFFFFFF))
_KS = (_K0, _K1, _K2)
_ROTS = ((13, 15, 26, 6), (17, 29, 16, 24))
_THRESH = np.uint32(838861 << 9)


def _rotl(x, r):
    return lax.shift_left(x, np.int32(r)) | lax.shift_right_logical(
        x, np.int32(32 - r))


def _cipher_bits(idx):
    x0 = _K0
    x1 = idx + _K1
    for i in range(5):
        for r in _ROTS[i % 2]:
            x0 = x0 + x1
            x1 = _rotl(x1, r) ^ x0
        x0 = x0 + _KS[(i + 1) % 3]
        x1 = x1 + _KS[(i + 2) % 3] + np.int32(i + 1)
    return x0 ^ x1


def _dropout_block(v_ref, o_ref):
    pid = pl.program_id(0)
    base = pid * np.int32(_BLOCK_ROWS * _LANES)
    idx = (base
           + lax.broadcasted_iota(jnp.int32, (_BLOCK_ROWS, _LANES), 0)
           * np.int32(_LANES)
           + lax.broadcasted_iota(jnp.int32, (_BLOCK_ROWS, _LANES), 1))
    bits = _cipher_bits(idx)
    keep = lax.bitcast_convert_type(bits, jnp.uint32) >= _THRESH
    o_ref[...] = jnp.where(keep, v_ref[...] * _INV_KEEP, np.float32(0.0))


def kernel(indices, values):
    v2d = values.reshape(_ROWS, _LANES)
    out = pl.pallas_call(
        _dropout_block,
        grid=(_GRID,),
        in_specs=[pl.BlockSpec((_BLOCK_ROWS, _LANES), lambda i: (i, 0))],
        out_specs=pl.BlockSpec((_BLOCK_ROWS, _LANES), lambda i: (i, 0)),
        out_shape=jax.ShapeDtypeStruct((_ROWS, _LANES), jnp.float32),
    )(v2d)
    return indices, out.reshape(_N)
